# Initial kernel scaffold; baseline (speedup 1.0000x reference)
import jax
import jax.numpy as jnp
from jax.experimental import pallas as pl


def kernel(node_features, node_attrs, edge_index, edge_attrs, edge_embedding, W_lin1, W_mlp0, W_mlp1, W_lin2, W_sc):
    raise NotImplementedError("write your pallas kernel here")


# speedup vs baseline: 2.4043x; 2.4043x over previous
import functools
import math

import jax
import jax.numpy as jnp
from jax import lax
from jax.experimental import pallas as pl
from jax.experimental.pallas import tpu as pltpu
from jax.experimental.pallas import tpu_sc as plsc

_N = 10000
_E = 320000
_D = 128
_D_ATTR = 16
_D_EMB = 16
_HID = 8
_AVG = 32.0

_NC = 2
_NS = 16
_NW = _NC * _NS
_B = 128
_NCHUNK = _E // _B
_NPAD = 10240
_STRIPE = _NPAD // _NS


def _edge_weight_body(emb_ref, ea_ref, wm0_ref, wm1_ref, out_ref):
    z = jnp.dot(emb_ref[...], wm0_ref[...],
                preferred_element_type=jnp.float32) * (1.0 / math.sqrt(_D_EMB))
    h = z / (1.0 + jnp.exp(-z))
    w = jnp.dot(h, wm1_ref[...], preferred_element_type=jnp.float32)
    out_ref[...] = w * ea_ref[...] * (1.0 / (math.sqrt(_HID) * math.sqrt(_AVG)))


def _edge_weights(emb, ea, wm0, wm1):
    be = 6400
    grid = _E // be
    return pl.pallas_call(
        _edge_weight_body,
        grid=(grid,),
        in_specs=[
            pl.BlockSpec((be, _D_EMB), lambda i: (i, 0)),
            pl.BlockSpec((be, 1), lambda i: (i, 0)),
            pl.BlockSpec((_D_EMB, _HID), lambda i: (0, 0)),
            pl.BlockSpec((_HID, _D), lambda i: (0, 0)),
        ],
        out_specs=pl.BlockSpec((be, _D), lambda i: (i, 0)),
        out_shape=jax.ShapeDtypeStruct((_E, _D), jnp.float32),
    )(emb, ea, wm0, wm1)


def _node_body(x_ref, attrs_ref, wlin1_ref, wsct_ref, xl_ref, sc_ref):
    x = x_ref[...]
    a = attrs_ref[...]
    xl_ref[...] = jnp.dot(x, wlin1_ref[...],
                          preferred_element_type=jnp.float32) * (1.0 / math.sqrt(_D))
    acc = jnp.zeros_like(x)
    for j in range(_D_ATTR):
        acc = acc + jnp.dot(x * a[:, j:j + 1], wsct_ref[j],
                            preferred_element_type=jnp.float32)
    sc_ref[...] = acc * (1.0 / math.sqrt(_D * _D_ATTR))


def _node_side(x, attrs, wlin1, wsct):
    bn = 2000
    grid = _N // bn
    return pl.pallas_call(
        _node_body,
        grid=(grid,),
        in_specs=[
            pl.BlockSpec((bn, _D), lambda i: (i, 0)),
            pl.BlockSpec((bn, _D_ATTR), lambda i: (i, 0)),
            pl.BlockSpec((_D, _D), lambda i: (0, 0)),
            pl.BlockSpec((_D_ATTR, _D, _D), lambda i: (0, 0, 0)),
        ],
        out_specs=[
            pl.BlockSpec((bn, _D), lambda i: (i, 0)),
            pl.BlockSpec((bn, _D), lambda i: (i, 0)),
        ],
        out_shape=[
            jax.ShapeDtypeStruct((_N, _D), jnp.float32),
            jax.ShapeDtypeStruct((_N, _D), jnp.float32),
        ],
    )(x, attrs, wlin1, wsct)


def _final_body(p0_ref, p1_ref, sc_ref, wlin2_ref, out_ref):
    p = p0_ref[...] + p1_ref[...]
    out_ref[...] = jnp.dot(p, wlin2_ref[...],
                           preferred_element_type=jnp.float32) * (1.0 / math.sqrt(_D)) + sc_ref[...]


def _final(p0, p1, sc, wlin2):
    bn = 2000
    grid = _N // bn
    return pl.pallas_call(
        _final_body,
        grid=(grid,),
        in_specs=[
            pl.BlockSpec((bn, _D), lambda i: (i, 0)),
            pl.BlockSpec((bn, _D), lambda i: (i, 0)),
            pl.BlockSpec((bn, _D), lambda i: (i, 0)),
            pl.BlockSpec((_D, _D), lambda i: (0, 0)),
        ],
        out_specs=pl.BlockSpec((bn, _D), lambda i: (i, 0)),
        out_shape=jax.ShapeDtypeStruct((_N, _D), jnp.float32),
    )(p0, p1, sc, wlin2)


def _sc_body(xl_hbm, wcomb_hbm, src_hbm, dst_hbm, zeros_hbm, out_hbm,
             srcv, dstv, rows, wv, acc, sem_g, sem_w):
    c = lax.axis_index("c")
    s = lax.axis_index("s")
    wid = c * _NS + s

    pltpu.sync_copy(zeros_hbm.at[pl.ds(s * _STRIPE, _STRIPE)],
                    acc.at[pl.ds(s * _STRIPE, _STRIPE)])
    plsc.subcore_barrier()

    nt = (_NCHUNK - wid + _NW - 1) // _NW

    def chunk_body(t, carry):
        base = (wid + t * _NW) * _B
        pltpu.sync_copy(src_hbm.at[pl.ds(base, _B)], srcv)
        pltpu.sync_copy(dst_hbm.at[pl.ds(base, _B)], dstv)
        cp_g = pltpu.async_copy(xl_hbm.at[srcv], rows, sem_g)
        cp_w = pltpu.async_copy(wcomb_hbm.at[pl.ds(base, _B)], wv, sem_w)
        cp_w.wait()
        cp_g.wait()

        def mul_i(i, carry2):
            for j in range(_D // 16):
                rows[i, pl.ds(j * 16, 16)] = (rows[i, pl.ds(j * 16, 16)]
                                              * wv[i, pl.ds(j * 16, 16)])
            return carry2

        lax.fori_loop(0, _B, mul_i, 0)
        pltpu.sync_copy(rows, acc.at[dstv], add=True)
        return carry

    lax.fori_loop(0, nt, chunk_body, 0)
    plsc.subcore_barrier()

    pltpu.sync_copy(acc.at[pl.ds(s * _STRIPE, _STRIPE)],
                    out_hbm.at[pl.ds(c * _NPAD + s * _STRIPE, _STRIPE)])


def _sc_scatter(xl, wcomb, src, dst, zeros):
    mesh = plsc.VectorSubcoreMesh(core_axis_name="c", subcore_axis_name="s")
    f = functools.partial(
        pl.kernel,
        mesh=mesh,
        out_type=jax.ShapeDtypeStruct((_NC * _NPAD, _D), jnp.float32),
        scratch_types=[
            pltpu.VMEM((_B,), jnp.int32),
            pltpu.VMEM((_B,), jnp.int32),
            pltpu.VMEM((_B, _D), jnp.float32),
            pltpu.VMEM((_B, _D), jnp.float32),
            pltpu.VMEM_SHARED((_NPAD, _D), jnp.float32),
            pltpu.SemaphoreType.DMA,
            pltpu.SemaphoreType.DMA,
        ],
    )(_sc_body)
    return f(xl, wcomb, src, dst, zeros)


def kernel(node_features, node_attrs, edge_index, edge_attrs, edge_embedding,
           W_lin1, W_mlp0, W_mlp1, W_lin2, W_sc):
    edge_src = edge_index[1]
    edge_dst = edge_index[0]
    wsct = jnp.transpose(W_sc, (1, 0, 2))
    zeros = jnp.zeros((_NPAD, _D), jnp.float32)

    wcomb = _edge_weights(edge_embedding, edge_attrs, W_mlp0, W_mlp1)
    xl, sc = _node_side(node_features, node_attrs, W_lin1, wsct)
    parts = _sc_scatter(xl, wcomb, edge_src, edge_dst, zeros)
    return _final(parts[:_N], parts[_NPAD:_NPAD + _N], sc, W_lin2)
